# Initial kernel scaffold; baseline (speedup 1.0000x reference)
import jax
import jax.numpy as jnp
from jax.experimental import pallas as pl


def kernel(pre_activation_o, img_idxs, head_idxs, grid_y_idxs, grid_x_idxs):
    raise NotImplementedError("write your pallas kernel here")


# speedup vs baseline: 1.1245x; 1.1245x over previous
import functools

import jax
import jax.numpy as jnp
from jax import lax
from jax.experimental import pallas as pl
from jax.experimental.pallas import tpu as pltpu
from jax.experimental.pallas import tpu_sc as plsc

_B, _H, _GY, _GX = 32, 3, 160, 160
_NTOT = _B * _H * _GY * _GX
_NA = 20000
_NC, _NS = 2, 16
_NW = _NC * _NS
_CHUNK = 128
_CPW = 5
_PER_W = _CHUNK * _CPW
_NPAD = _NW * _PER_W
_GROUPS = _PER_W // 16

_mesh = plsc.VectorSubcoreMesh(core_axis_name="c", subcore_axis_name="s")


@functools.partial(
    pl.kernel,
    mesh=_mesh,
    out_type=jax.ShapeDtypeStruct((_NTOT,), jnp.int32),
    scratch_types=[
        pltpu.VMEM((_CPW, _CHUNK), jnp.int32),
        pltpu.VMEM((_CPW, _CHUNK), jnp.int32),
        pltpu.SemaphoreType.DMA,
    ],
)
def _sc_scatter_ids(idx_hbm, table_hbm, idx_v, ids_v, sem):
    wid = lax.axis_index("s") * _NC + lax.axis_index("c")
    base = wid * _PER_W
    for j in range(_CPW):
        pltpu.sync_copy(idx_hbm.at[pl.ds(base + j * _CHUNK, _CHUNK)], idx_v.at[j])
    for g in range(_GROUPS):
        j, o = divmod(g * 16, _CHUNK)
        pos = base + g * 16 + lax.iota(jnp.int32, 16)
        ids_v[j, pl.ds(o, 16)] = jnp.where(pos < _NA, pos, 0)
    copies = [
        pltpu.async_copy(ids_v.at[j], table_hbm.at[idx_v.at[j]], sem)
        for j in range(_CPW)
    ]
    for cp in copies:
        cp.wait()


@functools.partial(
    pl.kernel,
    mesh=_mesh,
    out_type=jax.ShapeDtypeStruct((_NW * 16,), jnp.float32),
    scratch_types=[
        pltpu.VMEM((_CPW, _CHUNK), jnp.int32),
        pltpu.VMEM((_CPW, _CHUNK), jnp.int32),
        pltpu.VMEM((_CPW, _CHUNK), jnp.float32),
        pltpu.VMEM((16,), jnp.float32),
        pltpu.SemaphoreType.DMA,
    ],
)
def _sc_masked_sum(x_hbm, idx_hbm, table_hbm, out_hbm, idx_v, w_v, v_v, acc_v, sem):
    wid = lax.axis_index("s") * _NC + lax.axis_index("c")
    base = wid * _PER_W
    for j in range(_CPW):
        pltpu.sync_copy(idx_hbm.at[pl.ds(base + j * _CHUNK, _CHUNK)], idx_v.at[j])
    copies = [
        pltpu.async_copy(table_hbm.at[idx_v.at[j]], w_v.at[j], sem)
        for j in range(_CPW)
    ]
    copies += [
        pltpu.async_copy(x_hbm.at[idx_v.at[j]], v_v.at[j], sem)
        for j in range(_CPW)
    ]
    for cp in copies:
        cp.wait()
    acc = jnp.zeros((16,), jnp.float32)
    for g in range(_GROUPS):
        j, o = divmod(g * 16, _CHUNK)
        pos = base + g * 16 + lax.iota(jnp.int32, 16)
        slot_id = jnp.where(pos < _NA, pos, 0)
        win = (w_v[j, pl.ds(o, 16)] == slot_id) & (pos < _NA)
        acc = acc + jnp.where(win, v_v[j, pl.ds(o, 16)], 0.0)
    acc_v[...] = acc
    pltpu.sync_copy(acc_v, out_hbm.at[pl.ds(wid * 16, 16)])


def _tc_body(x_ref, out_ref):
    @pl.when(pl.program_id(0) == 0)
    def _init():
        out_ref[0, 0] = 0.0

    x = x_ref[...]
    f = jnp.maximum(x, 0.0) + jnp.log1p(jnp.exp(-jnp.abs(x)))
    out_ref[0, 0] += jnp.sum(f)


_TC_GRID = 8
_ROWS = _NTOT // 128

_tc_softplus_sum = pl.pallas_call(
    _tc_body,
    grid=(_TC_GRID,),
    in_specs=[pl.BlockSpec((_ROWS // _TC_GRID, 128), lambda i: (i, 0))],
    out_specs=pl.BlockSpec((1, 1), lambda i: (0, 0), memory_space=pltpu.SMEM),
    out_shape=jax.ShapeDtypeStruct((1, 1), jnp.float32),
)


def kernel(pre_activation_o, img_idxs, head_idxs, grid_y_idxs, grid_x_idxs):
    flat = (
        (img_idxs.astype(jnp.int32) * _H + head_idxs) * _GY + grid_y_idxs
    ) * _GX + grid_x_idxs
    pad = jnp.broadcast_to(flat[0], (_NPAD - _NA,))
    idx_pad = jnp.concatenate([flat, pad])
    dense = _tc_softplus_sum(pre_activation_o.reshape(_ROWS, 128))[0, 0]
    table = _sc_scatter_ids(idx_pad)
    partials = _sc_masked_sum(pre_activation_o.reshape(_NTOT), idx_pad, table)
    return (dense - jnp.sum(partials)) / _NTOT


# speedup vs baseline: 2.0964x; 1.8642x over previous
import functools

import jax
import jax.numpy as jnp
from jax import lax
from jax.experimental import pallas as pl
from jax.experimental.pallas import tpu as pltpu
from jax.experimental.pallas import tpu_sc as plsc

_B, _H, _GY, _GX = 32, 3, 160, 160
_NTOT = _B * _H * _GY * _GX
_NA = 20000
_NC, _NS = 2, 16
_NW = _NC * _NS
_CHUNK = 128
_CPW = 5
_PER_W = _CHUNK * _CPW
_NPAD = _NW * _PER_W
_GROUPS = _PER_W // 16

_mesh = plsc.VectorSubcoreMesh(core_axis_name="c", subcore_axis_name="s")


@functools.partial(
    pl.kernel,
    mesh=_mesh,
    out_type=jax.ShapeDtypeStruct((_NTOT,), jnp.int32),
    scratch_types=[
        pltpu.VMEM((_CPW, _CHUNK), jnp.int32),
        pltpu.VMEM((_CPW, _CHUNK), jnp.int32),
        pltpu.SemaphoreType.DMA,
    ],
)
def _sc_scatter_ids(idx_hbm, table_hbm, idx_v, ids_v, sem):
    wid = lax.axis_index("s") * _NC + lax.axis_index("c")
    base = wid * _PER_W
    for j in range(_CPW):
        pltpu.sync_copy(idx_hbm.at[pl.ds(base + j * _CHUNK, _CHUNK)], idx_v.at[j])
    for g in range(_GROUPS):
        j, o = divmod(g * 16, _CHUNK)
        pos = base + g * 16 + lax.iota(jnp.int32, 16)
        ids_v[j, pl.ds(o, 16)] = jnp.where(pos < _NA, pos, pos - _NA)
    copies = [
        pltpu.async_copy(ids_v.at[j], table_hbm.at[idx_v.at[j]], sem)
        for j in range(_CPW)
    ]
    for cp in copies:
        cp.wait()


@functools.partial(
    pl.kernel,
    mesh=_mesh,
    out_type=jax.ShapeDtypeStruct((_NW * 16,), jnp.float32),
    scratch_types=[
        pltpu.VMEM((_CPW, _CHUNK), jnp.int32),
        pltpu.VMEM((_CPW, _CHUNK), jnp.int32),
        pltpu.VMEM((_CPW, _CHUNK), jnp.float32),
        pltpu.VMEM((16,), jnp.float32),
        pltpu.SemaphoreType.DMA,
    ],
)
def _sc_masked_sum(x_hbm, idx_hbm, table_hbm, out_hbm, idx_v, w_v, v_v, acc_v, sem):
    wid = lax.axis_index("s") * _NC + lax.axis_index("c")
    base = wid * _PER_W
    for j in range(_CPW):
        pltpu.sync_copy(idx_hbm.at[pl.ds(base + j * _CHUNK, _CHUNK)], idx_v.at[j])
    copies = [
        pltpu.async_copy(table_hbm.at[idx_v.at[j]], w_v.at[j], sem)
        for j in range(_CPW)
    ]
    copies += [
        pltpu.async_copy(x_hbm.at[idx_v.at[j]], v_v.at[j], sem)
        for j in range(_CPW)
    ]
    for cp in copies:
        cp.wait()
    acc = jnp.zeros((16,), jnp.float32)
    for g in range(_GROUPS):
        j, o = divmod(g * 16, _CHUNK)
        pos = base + g * 16 + lax.iota(jnp.int32, 16)
        slot_id = jnp.where(pos < _NA, pos, pos - _NA)
        win = (w_v[j, pl.ds(o, 16)] == slot_id) & (pos < _NA)
        acc = acc + jnp.where(win, v_v[j, pl.ds(o, 16)], 0.0)
    acc_v[...] = acc
    pltpu.sync_copy(acc_v, out_hbm.at[pl.ds(wid * 16, 16)])


def _tc_body(x_ref, out_ref):
    @pl.when(pl.program_id(0) == 0)
    def _init():
        out_ref[0, 0] = 0.0

    x = x_ref[...]
    f = jnp.maximum(x, 0.0) + jnp.log1p(jnp.exp(-jnp.abs(x)))
    out_ref[0, 0] += jnp.sum(f)


_TC_GRID = 8
_ROWS = _NTOT // 128

_tc_softplus_sum = pl.pallas_call(
    _tc_body,
    grid=(_TC_GRID,),
    in_specs=[pl.BlockSpec((_ROWS // _TC_GRID, 128), lambda i: (i, 0))],
    out_specs=pl.BlockSpec((1, 1), lambda i: (0, 0), memory_space=pltpu.SMEM),
    out_shape=jax.ShapeDtypeStruct((1, 1), jnp.float32),
)


def kernel(pre_activation_o, img_idxs, head_idxs, grid_y_idxs, grid_x_idxs):
    flat = (
        (img_idxs.astype(jnp.int32) * _H + head_idxs) * _GY + grid_y_idxs
    ) * _GX + grid_x_idxs
    idx_pad = jnp.concatenate([flat, flat[: _NPAD - _NA]])
    dense = _tc_softplus_sum(pre_activation_o.reshape(_ROWS, 128))[0, 0]
    table = _sc_scatter_ids(idx_pad)
    partials = _sc_masked_sum(pre_activation_o.reshape(_NTOT), idx_pad, table)
    return (dense - jnp.sum(partials)) / _NTOT
